# baseline (speedup 1.0000x reference)
import functools
import jax
import jax.numpy as jnp
from jax import lax
from jax.experimental import pallas as pl
from jax.experimental.pallas import tpu as pltpu
from jax.experimental.pallas import tpu_sc as plsc

D = 64
L = 200
NC = 2
NS = 16
NW = NC * NS
BT = 128


NBUF = 4


def _body(ids_hbm, seg_hbm, tok_hbm, ps_hbm, out_hbm,
          idx0, seg0, tok0, outb0, idx1, seg1, tok1, outb1,
          idx2, seg2, tok2, outb2, idx3, seg3, tok3, outb3,
          semL0, semT0, semW0, semP0, semL1, semT1, semW1, semP1,
          semL2, semT2, semW2, semP2, semL3, semT3, semW3, semP3):
    wid = lax.axis_index("s") * NC + lax.axis_index("c")
    iota16 = lax.iota(jnp.int32, 16)

    sets = ((idx0, seg0, tok0, outb0, semL0, semT0, semW0, semP0),
            (idx1, seg1, tok1, outb1, semL1, semT1, semW1, semP1),
            (idx2, seg2, tok2, outb2, semL2, semT2, semW2, semP2),
            (idx3, seg3, tok3, outb3, semL3, semT3, semW3, semP3))

    def start_L(l, s):
        idx_r, seg_r, sem = sets[s][0], sets[s][1], sets[s][4]
        lc = lax.min(l, L - 1)
        pltpu.async_copy(ids_hbm.at[lc, pl.ds(wid * BT, BT)], idx_r, sem)
        pltpu.async_copy(seg_hbm.at[lc, pl.ds(wid * BT, BT)], seg_r, sem)

    def wait_L(s):
        idx_r, seg_r, sem = sets[s][0], sets[s][1], sets[s][4]
        pltpu.make_async_copy(ids_hbm.at[0, pl.ds(0, BT)], idx_r, sem).wait()
        pltpu.make_async_copy(seg_hbm.at[0, pl.ds(0, BT)], seg_r, sem).wait()

    def start_T(s):
        idx_r, tok_r, sem = sets[s][0], sets[s][2], sets[s][5]
        pltpu.async_copy(tok_hbm.at[idx_r], tok_r, sem, add=True)

    def wait_T(s):
        tok_r, sem = sets[s][2], sets[s][5]
        pltpu.make_async_copy(tok_hbm.at[pl.ds(0, BT)], tok_r, sem).wait()

    def compute_X(l, s):
        seg_r = sets[s][1]
        lc = lax.min(l, L - 1)
        for v in range(BT // 16):
            sl = pl.ds(v * 16, 16)
            seg_r[sl] = seg_r[sl] * L + lc

    def start_P(s):
        seg_r, tok_r, sem = sets[s][1], sets[s][2], sets[s][7]
        pltpu.async_copy(ps_hbm.at[seg_r], tok_r, sem)

    def wait_P(s):
        tok_r, sem = sets[s][2], sets[s][7]
        pltpu.make_async_copy(ps_hbm.at[pl.ds(0, BT)], tok_r, sem).wait()

    def main_C(s):
        tok_r, out_r = sets[s][2], sets[s][3]
        tokrows = [iota16 + (v * 16) for v in range(BT // 16)]

        def dbody(d0, carry):
            dcol = lax.rem(iota16 + d0, D)
            dcol128 = dcol * BT
            for v in range(BT // 16):
                a = plsc.load_gather(tok_r, [tokrows[v], dcol])
                plsc.store_scatter(out_r, [dcol128 + tokrows[v]], a)
            return carry
        lax.fori_loop(0, D, dbody, 0, unroll=8)

    def start_W(l, s):
        out_r, sem = sets[s][3], sets[s][6]
        for di in range(D // 8):
            pltpu.async_copy(out_r.at[pl.ds(di * 1024, 1024)],
                             out_hbm.at[l, di, wid], sem)

    def wait_W(s):
        out_r, sem = sets[s][3], sets[s][6]
        for di in range(D // 8):
            pltpu.make_async_copy(out_r.at[pl.ds(di * 1024, 1024)],
                                  out_hbm.at[0, di, 0], sem).wait()

    def early(s):
        wait_P(s)
        start_T(s)

    def late(lk, s, first):
        if not first:
            wait_W(s)
        wait_T(s)
        start_L(lk + NBUF, s)
        main_C(s)
        start_W(lk, s)
        wait_L(s); compute_X(lk + NBUF, s)
        start_P(s)

    for k in range(NBUF):
        start_L(k, k)
    for k in range(NBUF):
        wait_L(k); compute_X(k, k); start_P(k)
    early(0); early(1); early(2)

    def round_(b, first):
        late(b + 0, 0, first)
        early(3)
        late(b + 1, 1, first)
        early(0)
        late(b + 2, 2, first)
        early(1)
        late(b + 3, 3, first)
        early(2)

    round_(0, True)

    @pl.loop(NBUF, L, step=NBUF)
    def _steady(b):
        round_(b, False)

    wait_T(0); wait_T(1); wait_T(2)
    wait_P(3)
    for k in range(NBUF):
        wait_W(k)


@jax.jit
def _sc_call(ids_t, seg_t, token_table, posseg):
    mesh = plsc.VectorSubcoreMesh(core_axis_name="c", subcore_axis_name="s")
    f = pl.kernel(
        _body,
        out_type=jax.ShapeDtypeStruct((L, D // 8, NW, 8 * BT), jnp.float32),
        mesh=mesh,
        compiler_params=pltpu.CompilerParams(use_tc_tiling_on_sc=False,
                                             needs_layout_passes=False,
                                             disable_bounds_checks=True),
        scratch_types=(
            [pltpu.VMEM((BT,), jnp.int32),
             pltpu.VMEM((BT,), jnp.int32),
             pltpu.VMEM((BT, D), jnp.float32),
             pltpu.VMEM((D * BT,), jnp.float32)] * 4
            + [pltpu.SemaphoreType.DMA] * 16
        ),
    )
    return f(ids_t, seg_t, token_table, posseg)


def kernel(input_ids, segment_ids, token_table, seg_table, pe):
    B_, L_ = input_ids.shape
    ids_t = jnp.transpose(input_ids).astype(jnp.int32)
    seg_t = jnp.transpose(segment_ids).astype(jnp.int32)
    posseg = (seg_table[:, None, :] + pe[0, :L_][None, :, :]).reshape(3 * L_, D)
    out6 = _sc_call(ids_t, seg_t, token_table, posseg)
    return (out6.reshape(L_, D // 8, NW, 8, BT)
            .transpose(2, 4, 0, 1, 3).reshape(B_, L_, D))


# speedup vs baseline: 1.4681x; 1.4681x over previous
import functools
import jax
import jax.numpy as jnp
from jax import lax
from jax.experimental import pallas as pl
from jax.experimental.pallas import tpu as pltpu
from jax.experimental.pallas import tpu_sc as plsc

D = 64
L = 200
NC = 2
NS = 16
NW = NC * NS
BT = 128


def _body(ids_hbm, seg_hbm, tok_hbm, ps_hbm, out_hbm,
          posseg_v,
          idx0, seg0, tok0, outb0, idx1, seg1, tok1, outb1,
          semL0, semT0, semW0, semL1, semT1, semW1):
    wid = lax.axis_index("s") * NC + lax.axis_index("c")
    iota16 = lax.iota(jnp.int32, 16)

    pltpu.sync_copy(ps_hbm, posseg_v)

    sets = ((idx0, seg0, tok0, outb0, semL0, semT0, semW0),
            (idx1, seg1, tok1, outb1, semL1, semT1, semW1))

    def start_L(l, s):
        idx_r, seg_r, sem = sets[s][0], sets[s][1], sets[s][4]
        lc = lax.min(l, L - 1)
        pltpu.async_copy(ids_hbm.at[lc, pl.ds(wid * BT, BT)], idx_r, sem)
        pltpu.async_copy(seg_hbm.at[lc, pl.ds(wid * BT, BT)], seg_r, sem)

    def wait_L(s):
        idx_r, seg_r, sem = sets[s][0], sets[s][1], sets[s][4]
        pltpu.make_async_copy(ids_hbm.at[0, pl.ds(0, BT)], idx_r, sem).wait()
        pltpu.make_async_copy(seg_hbm.at[0, pl.ds(0, BT)], seg_r, sem).wait()

    def start_T(s):
        idx_r, tok_r, sem = sets[s][0], sets[s][2], sets[s][5]
        pltpu.async_copy(tok_hbm.at[idx_r], tok_r, sem)

    def wait_T(s):
        tok_r, sem = sets[s][2], sets[s][5]
        pltpu.make_async_copy(tok_hbm.at[pl.ds(0, BT)], tok_r, sem).wait()

    def prep_C(l, s):
        seg_r = sets[s][1]
        return [(seg_r[pl.ds(v * 16, 16)] * L + l) * D for v in range(BT // 16)]

    def main_C(psoffs, s):
        tok_r, out_r = sets[s][2], sets[s][3]
        tokrows = [iota16 + (v * 16) for v in range(BT // 16)]

        def dbody(d0, carry):
            dcol = lax.rem(iota16 + d0, D)
            dcol128 = dcol * BT
            for v in range(BT // 16):
                a = plsc.load_gather(tok_r, [tokrows[v], dcol])
                p = plsc.load_gather(posseg_v, [psoffs[v] + dcol])
                plsc.store_scatter(out_r, [dcol128 + tokrows[v]], a + p)
            return carry
        lax.fori_loop(0, D, dbody, 0, unroll=8)

    def start_W(l, s):
        out_r, sem = sets[s][3], sets[s][6]
        for di in range(D // 8):
            pltpu.async_copy(out_r.at[pl.ds(di * 1024, 1024)],
                             out_hbm.at[l, di, wid], sem)

    def wait_W(s):
        out_r, sem = sets[s][3], sets[s][6]
        for di in range(D // 8):
            pltpu.make_async_copy(out_r.at[pl.ds(di * 1024, 1024)],
                                  out_hbm.at[0, di, 0], sem).wait()

    start_L(0, 0)
    start_L(1, 1)
    wait_L(0); start_T(0)
    wait_L(1); start_T(1)
    wait_T(0); pr = prep_C(0, 0); start_L(2, 0); main_C(pr, 0)
    start_W(0, 0)
    wait_L(0); start_T(0)
    wait_T(1); pr = prep_C(1, 1); start_L(3, 1); main_C(pr, 1)
    start_W(1, 1)

    @pl.loop(2, L, step=2)
    def _steady(l):
        wait_L(1); start_T(1)
        wait_W(0)
        wait_T(0); p0 = prep_C(l, 0); start_L(l + 2, 0); main_C(p0, 0)
        start_W(l, 0)
        wait_L(0); start_T(0)
        wait_W(1)
        wait_T(1); p1 = prep_C(l + 1, 1); start_L(l + 3, 1); main_C(p1, 1)
        start_W(l + 1, 1)

    wait_T(0)
    wait_L(1)
    wait_W(0)
    wait_W(1)


@jax.jit
def _sc_call(ids_t, seg_t, token_table, posseg):
    mesh = plsc.VectorSubcoreMesh(core_axis_name="c", subcore_axis_name="s")
    f = pl.kernel(
        _body,
        out_type=jax.ShapeDtypeStruct((L, D // 8, NW, 8 * BT), jnp.float32),
        mesh=mesh,
        compiler_params=pltpu.CompilerParams(use_tc_tiling_on_sc=False,
                                             needs_layout_passes=False),
        scratch_types=[
            pltpu.VMEM((3 * L * D,), jnp.float32),
            pltpu.VMEM((BT,), jnp.int32),
            pltpu.VMEM((BT,), jnp.int32),
            pltpu.VMEM((BT, D), jnp.float32),
            pltpu.VMEM((D * BT,), jnp.float32),
            pltpu.VMEM((BT,), jnp.int32),
            pltpu.VMEM((BT,), jnp.int32),
            pltpu.VMEM((BT, D), jnp.float32),
            pltpu.VMEM((D * BT,), jnp.float32),
            pltpu.SemaphoreType.DMA,
            pltpu.SemaphoreType.DMA,
            pltpu.SemaphoreType.DMA,
            pltpu.SemaphoreType.DMA,
            pltpu.SemaphoreType.DMA,
            pltpu.SemaphoreType.DMA,
        ],
    )
    return f(ids_t, seg_t, token_table, posseg)


def kernel(input_ids, segment_ids, token_table, seg_table, pe):
    B_, L_ = input_ids.shape
    ids_t = jnp.transpose(input_ids).astype(jnp.int32)
    seg_t = jnp.transpose(segment_ids).astype(jnp.int32)
    posseg = (seg_table[:, None, :] + pe[0, :L_][None, :, :]).reshape(3 * L_ * D)
    out6 = _sc_call(ids_t, seg_t, token_table, posseg)
    return (out6.reshape(L_, D // 8, NW, 8, BT)
            .transpose(2, 4, 0, 1, 3).reshape(B_, L_, D))
